# baseline (speedup 1.0000x reference)
import functools

import jax
import jax.numpy as jnp
from jax.experimental import pallas as pl
from jax.experimental.pallas import tpu as pltpu
from jax.experimental.pallas import tpu_sc as plsc

T = 200
H = 256
D = 512

_SC_INFO = plsc.get_sparse_core_info()
_NW = _SC_INFO.num_cores * _SC_INFO.num_subcores
TP = 256
_BPW = TP // _NW


def _sc_gather_rows(table, idx):
    mesh = plsc.VectorSubcoreMesh(core_axis_name="c", subcore_axis_name="s")

    @functools.partial(
        pl.kernel, mesh=mesh,
        out_type=jax.ShapeDtypeStruct((TP, H), jnp.float32),
        scratch_types=[
            pltpu.VMEM((_BPW,), jnp.int32),
            pltpu.VMEM((_BPW, H), jnp.float32),
            pltpu.SemaphoreType.DMA,
        ],
    )
    def k(table_hbm, idx_hbm, out_hbm, idx_v, rows_v, sem):
        wid = (jax.lax.axis_index("s") * _SC_INFO.num_cores
               + jax.lax.axis_index("c"))
        base = wid * _BPW
        pltpu.sync_copy(idx_hbm.at[pl.ds(base, _BPW)], idx_v)
        pltpu.async_copy(table_hbm.at[idx_v], rows_v, sem).wait()
        pltpu.sync_copy(rows_v, out_hbm.at[pl.ds(base, _BPW)])

    return k(table, idx)

_DNT = (((1,), (1,)), ((), ()))


def _dot_t(x, w):
    return jax.lax.dot_general(x, w, _DNT, preferred_element_type=jnp.float32)


def _lstm_kernel(x_img_ref, cells_ref, emb_c_ref, w_ih_ref,
                 w_hh_ref, b_ih_ref, b_hh_ref, fc_w_ref, fc_b_ref, out_ref,
                 z_ref, hs_ref, w_hh_t_ref):
    w_hh_t_ref[:] = w_hh_ref[:].astype(jnp.bfloat16).T

    x_img = x_img_ref[0:T, :]
    cell_ids = cells_ref[:, 15:16]
    oh_cell = (jax.lax.broadcasted_iota(jnp.int32, (T, 8), 1)
               == cell_ids).astype(jnp.float32)
    x_cell = jnp.dot(oh_cell, emb_c_ref[:], preferred_element_type=jnp.float32)

    z = (_dot_t(x_img, w_ih_ref[:, 0:H])
         + _dot_t(x_cell, w_ih_ref[:, H:D])
         + b_ih_ref[:] + b_hh_ref[:])
    z_ref[:] = z.astype(jnp.bfloat16)

    h = jnp.zeros((1, H), jnp.float32)
    c = jnp.zeros((1, H), jnp.float32)
    for t in range(T):
        g = z_ref[t:t + 1, :] + jnp.dot(
            h.astype(jnp.bfloat16), w_hh_t_ref[:],
            preferred_element_type=jnp.float32)
        i = jax.nn.sigmoid(g[:, 0:H])
        f = jax.nn.sigmoid(g[:, H:2 * H])
        gg = jnp.tanh(g[:, 2 * H:3 * H])
        o = jax.nn.sigmoid(g[:, 3 * H:4 * H])
        c = f * c + i * gg
        h = o * jnp.tanh(c)
        hs_ref[t:t + 1, :] = h

    logits = _dot_t(hs_ref[:], fc_w_ref[:]) + fc_b_ref[:]
    out_ref[:] = jax.nn.sigmoid(logits)


@functools.partial(jax.jit, static_argnames=("interpret",))
def _run(imgs, cells, emb_indice, emb_cell, w_ih, w_hh, b_ih, b_hh, fc_w,
         fc_b, interpret=False):
    idx = jnp.pad(imgs[:, 15].astype(jnp.int32), (0, TP - T))
    x_img = _sc_gather_rows(emb_indice, idx)
    return pl.pallas_call(
        _lstm_kernel,
        out_shape=jax.ShapeDtypeStruct((T, 2), jnp.float32),
        scratch_shapes=[
            pltpu.VMEM((T, 4 * H), jnp.bfloat16),
            pltpu.VMEM((T, H), jnp.float32),
            pltpu.VMEM((H, 4 * H), jnp.bfloat16),
        ],
        interpret=interpret,
    )(x_img, cells, emb_cell, w_ih, w_hh, b_ih, b_hh, fc_w, fc_b)


def kernel(cells, imgs, emb_cell, emb_indice, W_ih, W_hh, b_ih, b_hh, fc_w,
           fc_b):
    emb_cell8 = jnp.pad(emb_cell, ((0, 3), (0, 0)))
    return _run(imgs.astype(jnp.int32), cells.astype(jnp.int32), emb_indice,
                emb_cell8, W_ih, W_hh,
                b_ih.reshape(1, 4 * H), b_hh.reshape(1, 4 * H), fc_w,
                fc_b.reshape(1, 2))


# speedup vs baseline: 1.4167x; 1.4167x over previous
import functools

import jax
import jax.numpy as jnp
from jax.experimental import pallas as pl
from jax.experimental.pallas import tpu as pltpu

T = 200
H = 256
D = 512

_DNT = (((1,), (1,)), ((), ()))


def _dot_t(x, w):
    return jax.lax.dot_general(x, w, _DNT, preferred_element_type=jnp.float32)


def _lstm_kernel(imgs_ref, cells_ref, emb_i_ref, emb_c_ref, w_ih_ref,
                 w_hh_ref, b_ih_ref, b_hh_ref, fc_w_ref, fc_b_ref, out_ref,
                 z_ref, hs_ref, w_hh_t_ref):
    w_hh_t_ref[:] = w_hh_ref[:].astype(jnp.bfloat16).T

    img_ids = imgs_ref[:, 15:16]
    cell_ids = cells_ref[:, 15:16]
    oh_img = (jax.lax.broadcasted_iota(jnp.int32, (T, 900), 1)
              == img_ids).astype(jnp.float32)
    oh_cell = (jax.lax.broadcasted_iota(jnp.int32, (T, 8), 1)
               == cell_ids).astype(jnp.float32)
    x_img = jnp.dot(oh_img, emb_i_ref[:], preferred_element_type=jnp.float32)
    x_cell = jnp.dot(oh_cell, emb_c_ref[:], preferred_element_type=jnp.float32)

    z = (_dot_t(x_img, w_ih_ref[:, 0:H])
         + _dot_t(x_cell, w_ih_ref[:, H:D])
         + b_ih_ref[:] + b_hh_ref[:])
    z_ref[:] = z.astype(jnp.bfloat16)

    h = jnp.zeros((1, H), jnp.float32)
    c = jnp.zeros((1, H), jnp.float32)
    for t in range(T):
        g = z_ref[t:t + 1, :] + jnp.dot(
            h.astype(jnp.bfloat16), w_hh_t_ref[:],
            preferred_element_type=jnp.float32)
        i = 0.5 * jnp.tanh(0.5 * g[:, 0:H]) + 0.5
        f = 0.5 * jnp.tanh(0.5 * g[:, H:2 * H]) + 0.5
        gg = jnp.tanh(g[:, 2 * H:3 * H])
        o = 0.5 * jnp.tanh(0.5 * g[:, 3 * H:4 * H]) + 0.5
        c = f * c + i * gg
        h = o * jnp.tanh(c)
        hs_ref[t:t + 1, :] = h

    logits = _dot_t(hs_ref[:], fc_w_ref[:]) + fc_b_ref[:]
    out_ref[:] = jax.nn.sigmoid(logits)


@functools.partial(jax.jit, static_argnames=("interpret",))
def _run(imgs, cells, emb_indice, emb_cell, w_ih, w_hh_t, b_ih, b_hh, fc_w,
         fc_b, interpret=False):
    return pl.pallas_call(
        _lstm_kernel,
        out_shape=jax.ShapeDtypeStruct((T, 2), jnp.float32),
        scratch_shapes=[
            pltpu.VMEM((T, 4 * H), jnp.bfloat16),
            pltpu.VMEM((T, H), jnp.float32),
            pltpu.VMEM((H, 4 * H), jnp.bfloat16),
        ],
        interpret=interpret,
    )(imgs, cells, emb_indice, emb_cell, w_ih, w_hh_t, b_ih, b_hh, fc_w,
      fc_b)


def kernel(cells, imgs, emb_cell, emb_indice, W_ih, W_hh, b_ih, b_hh, fc_w,
           fc_b):
    emb_cell8 = jnp.pad(emb_cell, ((0, 3), (0, 0)))
    return _run(imgs.astype(jnp.int32), cells.astype(jnp.int32), emb_indice,
                emb_cell8, W_ih, W_hh,
                b_ih.reshape(1, 4 * H), b_hh.reshape(1, 4 * H), fc_w,
                fc_b.reshape(1, 2))


# speedup vs baseline: 1.4630x; 1.0327x over previous
import functools

import jax
import jax.numpy as jnp
from jax.experimental import pallas as pl
from jax.experimental.pallas import tpu as pltpu

T = 200
H = 256
D = 512

_DNT = (((1,), (1,)), ((), ()))


def _dot_t(x, w):
    return jax.lax.dot_general(x, w, _DNT, preferred_element_type=jnp.float32)


def _lstm_kernel(imgs_ref, cells_ref, emb_i_ref, emb_c_ref, w_ih_ref,
                 w_hh_ref, b_ih_ref, b_hh_ref, fc_w_ref, fc_b_ref, out_ref,
                 z_ref, hs_ref, w_hh_t_ref):
    w_hh_t_ref[:] = w_hh_ref[:].astype(jnp.bfloat16).T

    img_ids = imgs_ref[:, 15:16]
    cell_ids = cells_ref[:, 15:16]
    oh_img = (jax.lax.broadcasted_iota(jnp.int32, (T, 900), 1)
              == img_ids).astype(jnp.float32)
    oh_cell = (jax.lax.broadcasted_iota(jnp.int32, (T, 8), 1)
               == cell_ids).astype(jnp.float32)
    x_img = jnp.dot(oh_img, emb_i_ref[:], preferred_element_type=jnp.float32)
    emb_c8 = jnp.pad(emb_c_ref[:], ((0, 3), (0, 0)))
    x_cell = jnp.dot(oh_cell, emb_c8, preferred_element_type=jnp.float32)

    z = (_dot_t(x_img, w_ih_ref[:, 0:H])
         + _dot_t(x_cell, w_ih_ref[:, H:D])
         + b_ih_ref[:] + b_hh_ref[:])
    z_ref[:] = z.astype(jnp.bfloat16)

    h = jnp.zeros((1, H), jnp.float32)
    c = jnp.zeros((1, H), jnp.float32)
    for t in range(T):
        g = z_ref[t:t + 1, :] + jnp.dot(
            h.astype(jnp.bfloat16), w_hh_t_ref[:],
            preferred_element_type=jnp.float32)
        i = 0.5 * jnp.tanh(0.5 * g[:, 0:H]) + 0.5
        f = 0.5 * jnp.tanh(0.5 * g[:, H:2 * H]) + 0.5
        gg = jnp.tanh(g[:, 2 * H:3 * H])
        o = 0.5 * jnp.tanh(0.5 * g[:, 3 * H:4 * H]) + 0.5
        c = f * c + i * gg
        h = o * jnp.tanh(c)
        hs_ref[t:t + 1, :] = h

    logits = _dot_t(hs_ref[:], fc_w_ref[:]) + fc_b_ref[:]
    out_ref[:] = jax.nn.sigmoid(logits)


@functools.partial(jax.jit, static_argnames=("interpret",))
def _run(imgs, cells, emb_indice, emb_cell, w_ih, w_hh_t, b_ih, b_hh, fc_w,
         fc_b, interpret=False):
    return pl.pallas_call(
        _lstm_kernel,
        out_shape=jax.ShapeDtypeStruct((T, 2), jnp.float32),
        scratch_shapes=[
            pltpu.VMEM((T, 4 * H), jnp.bfloat16),
            pltpu.VMEM((T, H), jnp.float32),
            pltpu.VMEM((H, 4 * H), jnp.bfloat16),
        ],
        interpret=interpret,
    )(imgs, cells, emb_indice, emb_cell, w_ih, w_hh_t, b_ih, b_hh, fc_w,
      fc_b)


def kernel(cells, imgs, emb_cell, emb_indice, W_ih, W_hh, b_ih, b_hh, fc_w,
           fc_b):
    return _run(imgs.astype(jnp.int32), cells.astype(jnp.int32), emb_indice,
                emb_cell, W_ih, W_hh,
                b_ih.reshape(1, 4 * H), b_hh.reshape(1, 4 * H), fc_w,
                fc_b.reshape(1, 2))
